# baseline (speedup 1.0000x reference)
import functools

import jax
import jax.numpy as jnp
import numpy as np
from jax import lax
from jax.experimental import pallas as pl
from jax.experimental.pallas import tpu as pltpu

NN_ = 1024
NE_ = 2048
INC_ = 256
HID_ = 256
NH_ = 4
DH_ = HID_ // NH_
NL_ = 3
CH_ = 64
STR_ = 56
WIN_ = 128
WSTR_ = 120
PAD_ = 128
NP_ = NN_ + PAD_
NEG_INF = float("-inf")


def _erf(z):
    a1, a2, a3, a4, a5 = (0.254829592, -0.284496736, 1.421413741,
                          -1.453152027, 1.061405429)
    p = 0.3275911
    s = jnp.sign(z)
    za = jnp.abs(z)
    t = 1.0 / (1.0 + p * za)
    poly = ((((a5 * t + a4) * t + a3) * t + a2) * t + a1) * t
    y = 1.0 - poly * jnp.exp(-za * za)
    return s * y


def _gelu(y):
    return 0.5 * y * (1.0 + _erf(y * np.float32(1.0 / np.sqrt(2.0))))


def _dot(a, b):
    return jnp.dot(a, b, preferred_element_type=jnp.float32)


def _dot_nt(a, b):
    return lax.dot_general(a, b, (((1,), (1,)), ((), ())),
                           preferred_element_type=jnp.float32)


def _dot_tn(a, b):
    return lax.dot_general(a, b, (((0,), (0,)), ((), ())),
                           preferred_element_type=jnp.float32)


def _body(x_ref, src_ref, dst_ref, w_int_ref, b_in_ref,
          wq_ref, bq_ref, wkv_ref, bkv_ref,
          woc_ref, wc1_ref, bcp_ref, lnw_ref, lnb_ref,
          out_ref, pi_ref, app_ref, qpc_ref, kvp_ref, hp_ref):
    f32 = jnp.float32
    bf16 = jnp.bfloat16

    pi_ref[NN_:NP_, :] = jnp.zeros((PAD_, NN_), f32)
    app_ref[NN_:NP_, :] = jnp.zeros((PAD_, NN_), bf16)
    qpc_ref[NN_:NP_, :] = jnp.zeros((PAD_, 2 * HID_), f32)
    kvp_ref[NN_:NP_, :] = jnp.zeros((PAD_, 2 * HID_), f32)
    hp_ref[NN_:NP_, :] = jnp.zeros((PAD_, HID_), f32)

    iota_n = lax.broadcasted_iota(jnp.int32, (NN_, NE_), 0)
    srcmask = (iota_n == src_ref[0:1, :]).astype(bf16)
    dstmask = (iota_n == dst_ref[0:1, :]).astype(bf16)
    A = _dot_nt(dstmask, srcmask)
    A_bf = A.astype(bf16)
    ones_col = jnp.ones((NN_, 1), dtype=f32)
    indeg0 = _dot(A, ones_col)

    r_i = lax.broadcasted_iota(jnp.int32, (NN_, NN_), 0)
    c_i = lax.broadcasted_iota(jnp.int32, (NN_, NN_), 1)
    eye = (r_i == c_i).astype(f32)

    def one_wave(c):
        t, indeg, level = c
        cur = (indeg == 0.0) & (level == NN_)
        level = jnp.where(cur, t, level)
        dec = _dot(A, cur.astype(f32))
        return t + 1, indeg - dec, level

    def lvl_cond(c):
        t, indeg, level = c
        cur = (indeg == 0.0) & (level == NN_)
        return (t < NN_) & (jnp.max(cur.astype(jnp.int32)) > 0)

    def lvl_body(c):
        c = one_wave(c)
        c = one_wave(c)
        c = one_wave(c)
        return one_wave(c)

    level0 = jnp.full((NN_, 1), NN_, dtype=jnp.int32)
    _, _, level = lax.while_loop(lvl_cond, lvl_body,
                                 (jnp.int32(0), indeg0, level0))
    lmax = jnp.max(jnp.where(level < NN_, level, -1))

    lvl_f = level.astype(f32)
    lvl_row = _dot_tn(lvl_f, eye)
    before = (lvl_row < lvl_f) | ((lvl_row == lvl_f) & (c_i < r_i))
    g = jnp.sum(before.astype(f32), axis=1, keepdims=True)
    g_row = _dot_tn(g, eye)
    pi = (r_i == g_row.astype(jnp.int32)).astype(f32)
    pi_bf = pi.astype(bf16)
    pi_ref[0:NN_, :] = pi

    ap1 = _dot(pi_bf, A_bf).astype(bf16)
    app_ref[0:NN_, :] = _dot_nt(ap1, pi_bf).astype(bf16)

    actmask = (level >= 1) & (level < NN_)
    act_off = jnp.sum((level == 0).astype(jnp.int32))
    n_act = jnp.sum(actmask.astype(jnp.int32))
    nwin = (n_act + (WSTR_ - 1)) // WSTR_

    h0 = _dot(x_ref[...], w_int_ref[...]) + b_in_ref[0:1, :]
    out_ref[:, 0:HID_] = h0
    hp_ref[0:NN_, :] = _dot(pi, h0)

    scale = np.float32(1.0 / np.sqrt(DH_))
    iota_ch = lax.broadcasted_iota(jnp.int32, (CH_, 1), 0)
    iota_win = lax.broadcasted_iota(jnp.int32, (WIN_, 1), 0)
    prev_orig = h0
    for l in range(NL_):
        prevp = hp_ref[0:NN_, :]
        q_all = _dot(prevp, wq_ref[l]) + bq_ref[l, 0:1, :]
        pc = _dot(prevp, wc1_ref[l]) + bcp_ref[l, 0:1, :]
        qpc_ref[0:NN_, :] = jnp.concatenate([q_all, pc], axis=1)
        kvp_ref[0:NN_, :] = _dot(prevp, wkv_ref[l]) + bkv_ref[l, 0:1, :]

        def level_body(t, carry, l=l):
            off = jnp.sum((level < t).astype(jnp.int32))
            cnt = jnp.sum((level == t).astype(jnp.int32))
            nchunks = (cnt + (STR_ - 1)) // STR_

            def chunk_body(j, carry2):
                lo = off + j * STR_
                base = pl.multiple_of((lo // 8) * 8, 8)
                idx = base + iota_ch - off
                rows_valid = ((idx >= j * STR_)
                              & (idx < jnp.minimum((j + 1) * STR_, cnt)))
                qpcs = qpc_ref[pl.ds(base, CH_), :]
                a_sel = app_ref[pl.ds(base, CH_), :].astype(f32)
                amask = a_sel > 0.0
                kv = kvp_ref[0:NN_, :]

                outs = []
                for hd in range(NH_):
                    sl = slice(hd * DH_, (hd + 1) * DH_)
                    s = _dot_nt(qpcs[:, sl], kv[:, sl]) * scale
                    m = jnp.max(jnp.where(amask, s, NEG_INF),
                                axis=1, keepdims=True)
                    e = jnp.where(amask, a_sel * jnp.exp(s - m), 0.0)
                    den = jnp.sum(e, axis=1, keepdims=True)
                    den = jnp.where(den > 0.0, den, 1.0)
                    outs.append(_dot(e, kv[:, HID_ + hd * DH_:
                                           HID_ + (hd + 1) * DH_]) / den)
                agg = jnp.concatenate(outs, axis=1)
                y = qpcs[:, HID_:] + _dot(agg, woc_ref[l])
                mu = jnp.mean(y, axis=1, keepdims=True)
                var = jnp.mean((y - mu) ** 2, axis=1, keepdims=True)
                y = (y - mu) * lax.rsqrt(var + 1e-5) * lnw_ref[l, 0:1, :] \
                    + lnb_ref[l, 0:1, :]
                y = _gelu(y)
                nkv = _dot(y, wkv_ref[l]) + bkv_ref[l, 0:1, :]

                hcur = hp_ref[pl.ds(base, CH_), :]
                hp_ref[pl.ds(base, CH_), :] = jnp.where(rows_valid, y, hcur)
                kvcur = kvp_ref[pl.ds(base, CH_), :]
                kvp_ref[pl.ds(base, CH_), :] = jnp.where(rows_valid, nkv,
                                                         kvcur)
                return carry2

            return lax.fori_loop(0, nchunks, chunk_body, carry)

        lax.fori_loop(1, lmax + 1, level_body, 0)

        def win_body(w, yacc):
            lo = act_off + w * WSTR_
            start = pl.multiple_of((lo // 8) * 8, 8)
            p = start + iota_win
            valid = ((p >= lo) & (p < lo + WSTR_)
                     & (p < act_off + n_act))
            piwin = pi_ref[pl.ds(start, WIN_), :]
            hwin = hp_ref[pl.ds(start, WIN_), :]
            hwin = jnp.where(valid, hwin, 0.0)
            piwin = jnp.where(valid, piwin, 0.0)
            return yacc + _dot_tn(piwin, hwin)

        yacc = lax.fori_loop(0, nwin, win_body,
                             jnp.zeros((NN_, HID_), f32))
        prev_orig = jnp.where(actmask, yacc, prev_orig)
        out_ref[:, (l + 1) * HID_:(l + 2) * HID_] = prev_orig


@jax.jit
def kernel(x, edge_index, W_in, b_in, attn_in_w, attn_in_b,
           attn_out_w, attn_out_b, comb_w, comb_b, ln_w, ln_b):
    H = HID_
    src = edge_index[0].astype(jnp.int32).reshape(1, NE_)
    dst = edge_index[1].astype(jnp.int32).reshape(1, NE_)
    w_int = W_in.T
    wq = attn_in_w[:, :H, :].transpose(0, 2, 1)
    wk = attn_in_w[:, H:2 * H, :].transpose(0, 2, 1)
    wv = attn_in_w[:, 2 * H:, :].transpose(0, 2, 1)
    wkv = jnp.concatenate([wk, wv], axis=2)
    bq = attn_in_b[:, :H].reshape(NL_, 1, H)
    bkv = attn_in_b[:, H:].reshape(NL_, 1, 2 * H)
    wc1 = comb_w[:, :, :H].transpose(0, 2, 1)
    wc2 = comb_w[:, :, H:].transpose(0, 2, 1)
    woc = jnp.einsum('lij,ljk->lik', attn_out_w.transpose(0, 2, 1), wc2)
    bcp = (jnp.einsum('lj,ljk->lk', attn_out_b, wc2)
           + comb_b).reshape(NL_, 1, H)
    lnw = ln_w.reshape(NL_, 1, H)
    lnb = ln_b.reshape(NL_, 1, H)

    return pl.pallas_call(
        _body,
        out_shape=jax.ShapeDtypeStruct((NN_, (NL_ + 1) * H), jnp.float32),
        scratch_shapes=[
            pltpu.VMEM((NP_, NN_), jnp.float32),
            pltpu.VMEM((NP_, NN_), jnp.bfloat16),
            pltpu.VMEM((NP_, 2 * H), jnp.float32),
            pltpu.VMEM((NP_, 2 * H), jnp.float32),
            pltpu.VMEM((NP_, H), jnp.float32),
        ],
    )(x, src, dst, w_int, b_in.reshape(1, H),
      wq, bq, wkv, bkv, woc, wc1, bcp, lnw, lnb)


# speedup vs baseline: 1.0497x; 1.0497x over previous
import functools

import jax
import jax.numpy as jnp
import numpy as np
from jax import lax
from jax.experimental import pallas as pl
from jax.experimental.pallas import tpu as pltpu

NN_ = 1024
NE_ = 2048
INC_ = 256
HID_ = 256
NH_ = 4
DH_ = HID_ // NH_
NL_ = 3
CH_ = 64
STR_ = 56
WIN_ = 128
WSTR_ = 120
PAD_ = 128
NP_ = NN_ + PAD_
NEG_INF = float("-inf")


def _erf(z):
    a1, a2, a3, a4, a5 = (0.254829592, -0.284496736, 1.421413741,
                          -1.453152027, 1.061405429)
    p = 0.3275911
    s = jnp.sign(z)
    za = jnp.abs(z)
    t = 1.0 / (1.0 + p * za)
    poly = ((((a5 * t + a4) * t + a3) * t + a2) * t + a1) * t
    y = 1.0 - poly * jnp.exp(-za * za)
    return s * y


def _gelu(y):
    return 0.5 * y * (1.0 + _erf(y * np.float32(1.0 / np.sqrt(2.0))))


def _dot(a, b):
    return jnp.dot(a, b, preferred_element_type=jnp.float32)


def _dot_nt(a, b):
    return lax.dot_general(a, b, (((1,), (1,)), ((), ())),
                           preferred_element_type=jnp.float32)


def _dot_tn(a, b):
    return lax.dot_general(a, b, (((0,), (0,)), ((), ())),
                           preferred_element_type=jnp.float32)


def _body(x_ref, src_ref, dst_ref, w_int_ref, b_in_ref,
          wall_ref, ball_ref, wkv_ref, bkv_ref,
          woc_ref, lnw_ref, lnb_ref,
          out_ref, pi_ref, app_ref, qpc_ref, kvp_ref, hp_ref):
    f32 = jnp.float32
    bf16 = jnp.bfloat16

    pi_ref[NN_:NP_, :] = jnp.zeros((PAD_, NN_), f32)
    app_ref[NN_:NP_, :] = jnp.zeros((PAD_, NN_), bf16)
    qpc_ref[NN_:NP_, :] = jnp.zeros((PAD_, 2 * HID_), f32)
    kvp_ref[NN_:NP_, :] = jnp.zeros((PAD_, 2 * HID_), f32)
    hp_ref[NN_:NP_, :] = jnp.zeros((PAD_, HID_), f32)

    iota_n = lax.broadcasted_iota(jnp.int32, (NN_, NE_), 0)
    srcmask = (iota_n == src_ref[0:1, :]).astype(bf16)
    dstmask = (iota_n == dst_ref[0:1, :]).astype(bf16)
    A = _dot_nt(dstmask, srcmask)
    A_bf = A.astype(bf16)
    ones_col = jnp.ones((NN_, 1), dtype=bf16)
    indeg0 = _dot(A_bf, ones_col)

    r_i = lax.broadcasted_iota(jnp.int32, (NN_, NN_), 0)
    c_i = lax.broadcasted_iota(jnp.int32, (NN_, NN_), 1)
    eye = (r_i == c_i).astype(f32)

    def one_wave(c):
        t, indeg, level = c
        cur = (indeg == 0.0) & (level == NN_)
        level = jnp.where(cur, t, level)
        dec = _dot(A_bf, cur.astype(bf16))
        return t + 1, indeg - dec, level

    def lvl_cond(c):
        t, indeg, level = c
        cur = (indeg == 0.0) & (level == NN_)
        return (t < NN_) & (jnp.max(cur.astype(jnp.int32)) > 0)

    def lvl_body(c):
        c = one_wave(c)
        c = one_wave(c)
        c = one_wave(c)
        return one_wave(c)

    level0 = jnp.full((NN_, 1), NN_, dtype=jnp.int32)
    _, _, level = lax.while_loop(lvl_cond, lvl_body,
                                 (jnp.int32(0), indeg0, level0))
    lmax = jnp.max(jnp.where(level < NN_, level, -1))

    lvl_f = level.astype(f32)
    lvl_row = _dot_tn(lvl_f, eye)
    before = (lvl_row < lvl_f) | ((lvl_row == lvl_f) & (c_i < r_i))
    g = jnp.sum(before.astype(f32), axis=1, keepdims=True)
    g_row = _dot_tn(g, eye)
    pi = (r_i == g_row.astype(jnp.int32)).astype(f32)
    pi_bf = pi.astype(bf16)
    pi_ref[0:NN_, :] = pi

    ap1 = _dot(pi_bf, A_bf).astype(bf16)
    app_ref[0:NN_, :] = _dot_nt(ap1, pi_bf).astype(bf16)

    actmask = (level >= 1) & (level < NN_)
    act_off = jnp.sum((level == 0).astype(jnp.int32))
    n_act = jnp.sum(actmask.astype(jnp.int32))
    nwin = (n_act + (WSTR_ - 1)) // WSTR_

    h0 = _dot(x_ref[...], w_int_ref[...]) + b_in_ref[0:1, :]
    out_ref[:, 0:HID_] = h0
    hp_ref[0:NN_, :] = _dot(pi, h0)

    scale = np.float32(1.0 / np.sqrt(DH_))
    iota_ch = lax.broadcasted_iota(jnp.int32, (CH_, 1), 0)
    iota_win = lax.broadcasted_iota(jnp.int32, (WIN_, 1), 0)
    prev_orig = h0
    for l in range(NL_):
        prevp = hp_ref[0:NN_, :]
        proj = _dot(prevp, wall_ref[l]) + ball_ref[l, 0:1, :]
        qpc_ref[0:NN_, :] = proj[:, 0:2 * HID_]
        kvp_ref[0:NN_, :] = proj[:, 2 * HID_:]

        def level_body(t, carry, l=l):
            off = jnp.sum((level < t).astype(jnp.int32))
            cnt = jnp.sum((level == t).astype(jnp.int32))
            nchunks = (cnt + (STR_ - 1)) // STR_

            def chunk_body(j, carry2):
                lo = off + j * STR_
                base = pl.multiple_of((lo // 8) * 8, 8)
                idx = base + iota_ch - off
                rows_valid = ((idx >= j * STR_)
                              & (idx < jnp.minimum((j + 1) * STR_, cnt)))
                qpcs = qpc_ref[pl.ds(base, CH_), :]

                def attn(P):
                    def f(_):
                        a_sel = app_ref[pl.ds(base, CH_),
                                        0:P].astype(f32)
                        amask = a_sel > 0.0
                        outs = []
                        for hd in range(NH_):
                            sl = slice(hd * DH_, (hd + 1) * DH_)
                            s = _dot_nt(qpcs[:, sl],
                                        kvp_ref[0:P, sl]) * scale
                            m = jnp.max(jnp.where(amask, s, NEG_INF),
                                        axis=1, keepdims=True)
                            e = jnp.where(amask,
                                          a_sel * jnp.exp(s - m), 0.0)
                            den = jnp.sum(e, axis=1, keepdims=True)
                            den = jnp.where(den > 0.0, den, 1.0)
                            outs.append(
                                _dot(e, kvp_ref[0:P, HID_ + hd * DH_:
                                                HID_ + (hd + 1) * DH_])
                                / den)
                        return jnp.concatenate(outs, axis=1)
                    return f

                agg = lax.cond(off <= 256, attn(256), attn(NN_),
                               0)
                y = qpcs[:, HID_:] + _dot(agg, woc_ref[l])
                mu = jnp.mean(y, axis=1, keepdims=True)
                var = jnp.mean((y - mu) ** 2, axis=1, keepdims=True)
                y = (y - mu) * lax.rsqrt(var + 1e-5) * lnw_ref[l, 0:1, :] \
                    + lnb_ref[l, 0:1, :]
                y = _gelu(y)
                nkv = _dot(y, wkv_ref[l]) + bkv_ref[l, 0:1, :]

                hcur = hp_ref[pl.ds(base, CH_), :]
                hp_ref[pl.ds(base, CH_), :] = jnp.where(rows_valid, y, hcur)
                kvcur = kvp_ref[pl.ds(base, CH_), :]
                kvp_ref[pl.ds(base, CH_), :] = jnp.where(rows_valid, nkv,
                                                         kvcur)
                return carry2

            return lax.fori_loop(0, nchunks, chunk_body, carry)

        lax.fori_loop(1, lmax + 1, level_body, 0)

        def win_body(w, yacc):
            lo = act_off + w * WSTR_
            start = pl.multiple_of((lo // 8) * 8, 8)
            p = start + iota_win
            valid = ((p >= lo) & (p < lo + WSTR_)
                     & (p < act_off + n_act))
            piwin = pi_ref[pl.ds(start, WIN_), :]
            hwin = hp_ref[pl.ds(start, WIN_), :]
            hwin = jnp.where(valid, hwin, 0.0)
            piwin = jnp.where(valid, piwin, 0.0)
            return yacc + _dot_tn(piwin, hwin)

        yacc = lax.fori_loop(0, nwin, win_body,
                             jnp.zeros((NN_, HID_), f32))
        prev_orig = jnp.where(actmask, yacc, prev_orig)
        out_ref[:, (l + 1) * HID_:(l + 2) * HID_] = prev_orig


@jax.jit
def kernel(x, edge_index, W_in, b_in, attn_in_w, attn_in_b,
           attn_out_w, attn_out_b, comb_w, comb_b, ln_w, ln_b):
    H = HID_
    src = edge_index[0].astype(jnp.int32).reshape(1, NE_)
    dst = edge_index[1].astype(jnp.int32).reshape(1, NE_)
    w_int = W_in.T
    wq = attn_in_w[:, :H, :].transpose(0, 2, 1)
    wk = attn_in_w[:, H:2 * H, :].transpose(0, 2, 1)
    wv = attn_in_w[:, 2 * H:, :].transpose(0, 2, 1)
    wkv = jnp.concatenate([wk, wv], axis=2)
    bq = attn_in_b[:, :H].reshape(NL_, 1, H)
    bkv = attn_in_b[:, H:].reshape(NL_, 1, 2 * H)
    wc1 = comb_w[:, :, :H].transpose(0, 2, 1)
    wc2 = comb_w[:, :, H:].transpose(0, 2, 1)
    woc = jnp.einsum('lij,ljk->lik', attn_out_w.transpose(0, 2, 1), wc2)
    bcp = (jnp.einsum('lj,ljk->lk', attn_out_b, wc2)
           + comb_b).reshape(NL_, 1, H)
    lnw = ln_w.reshape(NL_, 1, H)
    lnb = ln_b.reshape(NL_, 1, H)
    wall = jnp.concatenate([wq, wc1, wkv], axis=2)
    ball = jnp.concatenate([bq, bcp, bkv], axis=2)

    return pl.pallas_call(
        _body,
        out_shape=jax.ShapeDtypeStruct((NN_, (NL_ + 1) * H), jnp.float32),
        scratch_shapes=[
            pltpu.VMEM((NP_, NN_), jnp.float32),
            pltpu.VMEM((NP_, NN_), jnp.bfloat16),
            pltpu.VMEM((NP_, 2 * H), jnp.float32),
            pltpu.VMEM((NP_, 2 * H), jnp.float32),
            pltpu.VMEM((NP_, H), jnp.float32),
        ],
    )(x, src, dst, w_int, b_in.reshape(1, H),
      wall, ball, wkv, bkv, woc, lnw, lnb)
